# baseline (speedup 1.0000x reference)
import functools

import jax
import jax.numpy as jnp
from jax import lax
from jax.experimental import pallas as pl
from jax.experimental.pallas import tpu as pltpu
from jax.experimental.pallas import tpu_sc as plsc


def _body(xr_ref, x_ref, out_ref, acc_ref, *, nb, nc, inv):
    b = pl.program_id(0)
    c = pl.program_id(1)

    @pl.when((b == 0) & (c == 0))
    def _():
        acc_ref[...] = jnp.zeros_like(acc_ref)

    xb = x_ref[0]
    d0 = xr_ref[0, 0] - xb
    d1 = xr_ref[1, 0] - xb
    s0 = jnp.sum(d0 * d0)
    s1 = jnp.sum(d1 * d1)
    lane = lax.broadcasted_iota(jnp.int32, (1, 2), 1)
    acc_ref[pl.ds(b, 1), :] += jnp.where(lane == 0, s0, s1)

    @pl.when((b == nb - 1) & (c == nc - 1))
    def _():
        arr = acc_ref[...]
        idx = lax.broadcasted_iota(jnp.int32, arr.shape, 0)
        col = lax.broadcasted_iota(jnp.int32, arr.shape, 1)
        neg = float("-inf")
        m0 = jnp.max(jnp.where(col == 0, arr, neg))
        m1 = jnp.max(jnp.where(col == 1, arr, neg))
        j0 = jnp.max(jnp.where((col == 0) & (arr == m0), idx, -1))
        j1 = jnp.max(jnp.where((col == 1) & (arr == m1), idx, -1))
        s0t = jnp.sum(jnp.where(col == 0, arr, 0.0))
        s1t = jnp.sum(jnp.where(col == 1, arr, 0.0))
        d0t = jnp.sum(jnp.where((col == 0) & (idx == j1), arr, 0.0))
        d1t = jnp.sum(jnp.where((col == 1) & (idx == j0), arr, 0.0))
        loss = (s0t - d0t + s1t - d1t) * inv
        out_ref[...] = jnp.full((1, 1), loss, jnp.float32)


def _sc_dummy_body(in_hbm, out_hbm, in_v, out_v):
    cid = lax.axis_index("c")
    sid = lax.axis_index("s")

    @pl.when((cid == 0) & (sid == 0))
    def _():
        pltpu.sync_copy(in_hbm, in_v)
        lanes = lax.iota(jnp.int32, 16)
        v = in_v[0, :]

        def shuf(a, k):
            return a.at[lanes ^ k].get(mode="promise_in_bounds")

        for k in (1, 2, 4, 8):
            v = v + shuf(v, k)
        out_v[...] = v
        pltpu.sync_copy(out_v, out_hbm)


def _pick_chunk(c0, h, w, budget_bytes=3400000):
    best = 1
    for g in range(1, c0 + 1):
        if c0 % g == 0 and g * h * w * 4 <= budget_bytes:
            best = g
    return best


def kernel(xr, x):
    B, C0, H, W = x.shape
    N = C0 * H * W
    G = _pick_chunk(C0, H, W)
    C = C0 // G
    rem = int(B * 0.9)
    inv = 1.0 / (rem * N)

    loss = pl.pallas_call(
        functools.partial(_body, nb=B, nc=C, inv=inv),
        grid=(B, C),
        in_specs=[
            pl.BlockSpec((2, 1, G, H, W), lambda b, c: (0, b, c, 0, 0)),
            pl.BlockSpec((1, G, H, W), lambda b, c: (b, c, 0, 0)),
        ],
        out_specs=pl.BlockSpec((1, 1), lambda b, c: (0, 0)),
        out_shape=jax.ShapeDtypeStruct((1, 1), jnp.float32),
        scratch_shapes=[pltpu.VMEM((B, 2), jnp.float32)],
    )(xr, x)

    mesh = plsc.VectorSubcoreMesh(core_axis_name="c", subcore_axis_name="s")
    dummy = pl.kernel(
        _sc_dummy_body,
        mesh=mesh,
        out_type=jax.ShapeDtypeStruct((16,), jnp.float32),
        scratch_types=[
            pltpu.VMEM((1, 16), jnp.float32),
            pltpu.VMEM((16,), jnp.float32),
        ],
    )
    z16 = dummy(jnp.zeros((1, 16), jnp.float32))
    return loss[0, 0] + jnp.minimum(z16[0], 0.0)


# speedup vs baseline: 1.1058x; 1.1058x over previous
import jax
import jax.numpy as jnp
from jax import lax
from jax.experimental import pallas as pl
from jax.experimental.pallas import tpu as pltpu


def _body(xr_ref, x_ref, out_ref, acc_ref, *, nb, nc, inv):
    b = pl.program_id(0)
    c = pl.program_id(1)

    @pl.when((b == 0) & (c == 0))
    def _():
        acc_ref[...] = jnp.zeros_like(acc_ref)

    xb = x_ref[0]
    d0 = xr_ref[0, 0] - xb
    d1 = xr_ref[1, 0] - xb
    s0 = jnp.sum(d0 * d0)
    s1 = jnp.sum(d1 * d1)
    lane = lax.broadcasted_iota(jnp.int32, (1, 2), 1)
    acc_ref[pl.ds(b, 1), :] += jnp.where(lane == 0, s0, s1)

    @pl.when((b == nb - 1) & (c == nc - 1))
    def _():
        arr = acc_ref[...]
        idx = lax.broadcasted_iota(jnp.int32, arr.shape, 0)
        col = lax.broadcasted_iota(jnp.int32, arr.shape, 1)
        neg = float("-inf")
        m0 = jnp.max(jnp.where(col == 0, arr, neg))
        m1 = jnp.max(jnp.where(col == 1, arr, neg))
        j0 = jnp.max(jnp.where((col == 0) & (arr == m0), idx, -1))
        j1 = jnp.max(jnp.where((col == 1) & (arr == m1), idx, -1))
        s0t = jnp.sum(jnp.where(col == 0, arr, 0.0))
        s1t = jnp.sum(jnp.where(col == 1, arr, 0.0))
        d0t = jnp.sum(jnp.where((col == 0) & (idx == j1), arr, 0.0))
        d1t = jnp.sum(jnp.where((col == 1) & (idx == j0), arr, 0.0))
        loss = (s0t - d0t + s1t - d1t) * inv
        out_ref[...] = jnp.full((1, 1), loss, jnp.float32)


def _pick_chunk(c0, h, w, budget_bytes=7 * 1024 * 1024):
    best = 1
    for g in range(1, c0 + 1):
        if c0 % g == 0 and g * h * w * 4 <= budget_bytes:
            best = g
    return best


def kernel(xr, x):
    import functools

    B, C0, H, W = x.shape
    N = C0 * H * W
    G = _pick_chunk(C0, H, W)
    C = C0 // G
    rem = int(B * 0.9)
    inv = 1.0 / (rem * N)

    loss = pl.pallas_call(
        functools.partial(_body, nb=B, nc=C, inv=inv),
        grid=(B, C),
        in_specs=[
            pl.BlockSpec((2, 1, G, H, W), lambda b, c: (0, b, c, 0, 0)),
            pl.BlockSpec((1, G, H, W), lambda b, c: (b, c, 0, 0)),
        ],
        out_specs=pl.BlockSpec((1, 1), lambda b, c: (0, 0)),
        out_shape=jax.ShapeDtypeStruct((1, 1), jnp.float32),
        scratch_shapes=[pltpu.VMEM((B, 2), jnp.float32)],
    )(xr, x)
    return loss[0, 0]


# speedup vs baseline: 1.1137x; 1.0072x over previous
import jax
import jax.numpy as jnp
from jax import lax
from jax.experimental import pallas as pl
from jax.experimental.pallas import tpu as pltpu


def _body(xr0_ref, xr1_ref, x_ref, out_ref, acc_ref, *, nb, nc, inv):
    b = pl.program_id(0)
    c = pl.program_id(1)

    @pl.when((b == 0) & (c == 0))
    def _():
        acc_ref[...] = jnp.zeros_like(acc_ref)

    xb = x_ref[0]
    d0 = xr0_ref[0, 0] - xb
    d1 = xr1_ref[0, 0] - xb
    s0 = jnp.sum(d0 * d0)
    s1 = jnp.sum(d1 * d1)
    lane = lax.broadcasted_iota(jnp.int32, (1, 2), 1)
    acc_ref[pl.ds(b, 1), :] += jnp.where(lane == 0, s0, s1)

    @pl.when((b == nb - 1) & (c == nc - 1))
    def _():
        arr = acc_ref[...]
        idx = lax.broadcasted_iota(jnp.int32, arr.shape, 0)
        col = lax.broadcasted_iota(jnp.int32, arr.shape, 1)
        neg = float("-inf")
        m0 = jnp.max(jnp.where(col == 0, arr, neg))
        m1 = jnp.max(jnp.where(col == 1, arr, neg))
        j0 = jnp.max(jnp.where((col == 0) & (arr == m0), idx, -1))
        j1 = jnp.max(jnp.where((col == 1) & (arr == m1), idx, -1))
        s0t = jnp.sum(jnp.where(col == 0, arr, 0.0))
        s1t = jnp.sum(jnp.where(col == 1, arr, 0.0))
        d0t = jnp.sum(jnp.where((col == 0) & (idx == j1), arr, 0.0))
        d1t = jnp.sum(jnp.where((col == 1) & (idx == j0), arr, 0.0))
        loss = (s0t - d0t + s1t - d1t) * inv
        out_ref[...] = jnp.full((1, 1), loss, jnp.float32)


def _pick_chunk(c0, h, w, budget_bytes=3400000):
    best = 1
    for g in range(1, c0 + 1):
        if c0 % g == 0 and g * h * w * 4 <= budget_bytes:
            best = g
    return best


def kernel(xr, x):
    import functools

    B, C0, H, W = x.shape
    N = C0 * H * W
    G = _pick_chunk(C0, H, W)
    C = C0 // G
    rem = int(B * 0.9)
    inv = 1.0 / (rem * N)

    loss = pl.pallas_call(
        functools.partial(_body, nb=B, nc=C, inv=inv),
        grid=(B, C),
        in_specs=[
            pl.BlockSpec((1, 1, G, H, W), lambda b, c: (0, b, c, 0, 0)),
            pl.BlockSpec((1, 1, G, H, W), lambda b, c: (1, b, c, 0, 0)),
            pl.BlockSpec((1, G, H, W), lambda b, c: (b, c, 0, 0)),
        ],
        out_specs=pl.BlockSpec((1, 1), lambda b, c: (0, 0)),
        out_shape=jax.ShapeDtypeStruct((1, 1), jnp.float32),
        scratch_shapes=[pltpu.VMEM((B, 2), jnp.float32)],
    )(xr, xr, x)
    return loss[0, 0]


# speedup vs baseline: 1.1181x; 1.0039x over previous
import functools

import jax
import jax.numpy as jnp
from jax import lax
from jax.experimental import pallas as pl
from jax.experimental.pallas import tpu as pltpu


def _body(xr0_ref, xr1_ref, x_ref, out_ref, acc_ref, *, nb, nc, inv):
    b = pl.program_id(0)
    c = pl.program_id(1)

    @pl.when((b == 0) & (c == 0))
    def _():
        acc_ref[...] = jnp.zeros_like(acc_ref)

    xb = x_ref[0]
    d0 = xr0_ref[0, 0] - xb
    d1 = xr1_ref[0, 0] - xb
    s0 = jnp.sum(d0 * d0)
    s1 = jnp.sum(d1 * d1)
    lane = lax.broadcasted_iota(jnp.int32, (1, 2), 1)
    acc_ref[pl.ds(b, 1), :] += jnp.where(lane == 0, s0, s1)

    @pl.when((b == nb - 1) & (c == nc - 1))
    def _():
        arr = acc_ref[...]
        idx = lax.broadcasted_iota(jnp.int32, arr.shape, 0)
        col = lax.broadcasted_iota(jnp.int32, arr.shape, 1)
        neg = float("-inf")
        m0 = jnp.max(jnp.where(col == 0, arr, neg))
        m1 = jnp.max(jnp.where(col == 1, arr, neg))
        j0 = jnp.max(jnp.where((col == 0) & (arr == m0), idx, -1))
        j1 = jnp.max(jnp.where((col == 1) & (arr == m1), idx, -1))
        s0t = jnp.sum(jnp.where(col == 0, arr, 0.0))
        s1t = jnp.sum(jnp.where(col == 1, arr, 0.0))
        d0t = jnp.sum(jnp.where((col == 0) & (idx == j1), arr, 0.0))
        d1t = jnp.sum(jnp.where((col == 1) & (idx == j0), arr, 0.0))
        loss = (s0t - d0t + s1t - d1t) * inv
        out_ref[...] = jnp.full((1, 1), loss, jnp.float32)


def _pick_chunk(c0, h, w, budget_bytes=3400000):
    best = 1
    for g in range(1, c0 + 1):
        if c0 % g == 0 and g * h * w * 4 <= budget_bytes:
            best = g
    return best


def kernel(xr, x):
    B, C0, H, W = x.shape
    N = C0 * H * W
    G = _pick_chunk(C0, H, W)
    C = C0 // G
    rem = int(B * 0.9)
    inv = 1.0 / (rem * N)

    loss = pl.pallas_call(
        functools.partial(_body, nb=B, nc=C, inv=inv),
        grid=(B, C),
        in_specs=[
            pl.BlockSpec((1, 1, G, H, W), lambda b, c: (0, b, c, 0, 0)),
            pl.BlockSpec((1, 1, G, H, W), lambda b, c: (1, b, c, 0, 0)),
            pl.BlockSpec((1, G, H, W), lambda b, c: (b, c, 0, 0)),
        ],
        out_specs=pl.BlockSpec((1, 1), lambda b, c: (0, 0)),
        out_shape=jax.ShapeDtypeStruct((1, 1), jnp.float32),
        scratch_shapes=[pltpu.VMEM((B, 2), jnp.float32)],
    )(xr, xr, x)
    return loss[0, 0]
